# Initial kernel scaffold; baseline (speedup 1.0000x reference)
import jax
import jax.numpy as jnp
from jax.experimental import pallas as pl


def kernel(opcode_idx, a, operand, c_in, opcode_embed, W_in, b_in, Wr, br, W1, b1, W2, b2, Wh1, bh1, Wh2, bh2, Wf1, bf1, Wf2, bf2):
    raise NotImplementedError("write your pallas kernel here")


# speedup vs baseline: 1.7421x; 1.7421x over previous
import functools

import jax
import jax.numpy as jnp
from jax.experimental import pallas as pl
from jax.experimental.pallas import tpu as pltpu

B = 4096
D_MODEL = 64
NUM_TILES = 16
TOP_K = 4
D_FF = 128
N_OPS = 12
D_EMB = D_MODEL // 4

BT = 512


def _fused_body(op_ref, a_ref, operand_ref, c_ref, emb_ref, win_ref, bin_ref,
                wr_ref, br_ref, w1_ref, b1_ref, w2_ref, b2_ref,
                wh1_ref, bh1_ref, wh2_ref, bh2_ref,
                wf1_ref, bf1_ref, wf2_ref, bf2_ref,
                rb_ref, fl_ref, topi_ref, aux_ref,
                acc_imp, acc_load):
    i = pl.program_id(0)
    nblk = pl.num_programs(0)

    op_idx = op_ref[...]
    ids12 = jax.lax.broadcasted_iota(jnp.int32, (1, N_OPS), 1)
    onehot = (op_idx == ids12).astype(jnp.float32)
    op_emb = jnp.dot(onehot, emb_ref[...],
                     precision=jax.lax.Precision.HIGHEST,
                     preferred_element_type=jnp.float32)
    bits = jax.lax.broadcasted_iota(jnp.int32, (1, 8), 1)
    a_bits = ((a_ref[...] >> bits) & 1).astype(jnp.float32)
    o_bits = ((operand_ref[...] >> bits) & 1).astype(jnp.float32)
    c_f = c_ref[...].astype(jnp.float32)

    feats = jnp.concatenate([op_emb, a_bits, o_bits, c_f], axis=1)
    x = jnp.dot(feats, win_ref[...],
                preferred_element_type=jnp.float32) + bin_ref[...]

    logits = jnp.dot(x, wr_ref[...], preferred_element_type=jnp.float32) + br_ref[...]
    m = jnp.max(logits, axis=-1, keepdims=True)
    e = jnp.exp(logits - m)
    probs = e / jnp.sum(e, axis=-1, keepdims=True)

    ids16 = jax.lax.broadcasted_iota(jnp.int32, (BT, NUM_TILES), 1)
    work = probs
    topv = []
    topi = []
    hot = []
    for _ in range(TOP_K):
        v = jnp.max(work, axis=-1, keepdims=True)
        idx = jnp.min(jnp.where(work == v, ids16, NUM_TILES), axis=-1,
                      keepdims=True)
        oh = (ids16 == idx)
        topv.append(v)
        topi.append(idx)
        hot.append(oh)
        work = jnp.where(oh, -1.0, work)
    tsum = topv[0] + topv[1] + topv[2] + topv[3]
    gate_full = jnp.zeros((BT, NUM_TILES), jnp.float32)
    for k in range(TOP_K):
        gate_full = gate_full + jnp.where(hot[k], topv[k] / tsum, 0.0)
    topi_ref[...] = jnp.concatenate(topi, axis=1)

    out = jnp.zeros((BT, D_MODEL), jnp.float32)
    for t in range(NUM_TILES):
        h = jnp.maximum(
            jnp.dot(x, w1_ref[t], preferred_element_type=jnp.float32)
            + b1_ref[t:t + 1, :], 0.0)
        y = jnp.dot(h, w2_ref[t], preferred_element_type=jnp.float32) \
            + b2_ref[t:t + 1, :]
        out = out + gate_full[:, t:t + 1] * y

    h1 = jnp.maximum(jnp.dot(out, wh1_ref[...], preferred_element_type=jnp.float32)
                     + bh1_ref[...], 0.0)
    rb_ref[...] = jax.nn.sigmoid(
        jnp.dot(h1, wh2_ref[...], preferred_element_type=jnp.float32) + bh2_ref[...])
    f1 = jnp.maximum(jnp.dot(out, wf1_ref[...], preferred_element_type=jnp.float32)
                     + bf1_ref[...], 0.0)
    fl_ref[...] = jax.nn.sigmoid(
        jnp.dot(f1, wf2_ref[...], preferred_element_type=jnp.float32) + bf2_ref[...])

    @pl.when(i == 0)
    def _init():
        acc_imp[...] = jnp.zeros((1, NUM_TILES), jnp.float32)
        acc_load[...] = jnp.zeros((1, NUM_TILES), jnp.float32)

    acc_imp[...] += jnp.sum(probs, axis=0, keepdims=True)
    acc_load[...] += jnp.sum((gate_full > 0).astype(jnp.float32), axis=0,
                             keepdims=True)

    @pl.when(i == nblk - 1)
    def _fin():
        imp = acc_imp[...] / B
        load = acc_load[...] / B
        aux_ref[0, 0] = NUM_TILES * jnp.sum(imp * load)


def kernel(opcode_idx, a, operand, c_in, opcode_embed, W_in, b_in, Wr, br,
           W1, b1, W2, b2, Wh1, bh1, Wh2, bh2, Wf1, bf1, Wf2, bf2):
    n = opcode_idx.shape[0]
    nblk = n // BT
    tok = lambda: pl.BlockSpec((BT, 1), lambda i: (i, 0))
    rep = lambda *shape: pl.BlockSpec(shape, lambda i: tuple(0 for _ in shape))

    grid_spec = pltpu.PrefetchScalarGridSpec(
        num_scalar_prefetch=0,
        grid=(nblk,),
        in_specs=[
            tok(), tok(), tok(), tok(),
            rep(N_OPS, D_EMB),
            rep(33, D_MODEL), rep(1, D_MODEL),
            rep(D_MODEL, NUM_TILES), rep(1, NUM_TILES),
            rep(NUM_TILES, D_MODEL, D_FF), rep(NUM_TILES, D_FF),
            rep(NUM_TILES, D_FF, D_MODEL), rep(NUM_TILES, D_MODEL),
            rep(D_MODEL, 64), rep(1, 64),
            rep(64, 8), rep(1, 8),
            rep(D_MODEL, 32), rep(1, 32),
            rep(32, 4), rep(1, 4),
        ],
        out_specs=[
            pl.BlockSpec((BT, 8), lambda i: (i, 0)),
            pl.BlockSpec((BT, 4), lambda i: (i, 0)),
            pl.BlockSpec((BT, 4), lambda i: (i, 0)),
            pl.BlockSpec(memory_space=pltpu.SMEM),
        ],
        scratch_shapes=[
            pltpu.VMEM((1, NUM_TILES), jnp.float32),
            pltpu.VMEM((1, NUM_TILES), jnp.float32),
        ],
    )
    rb, fl, ti, aux = pl.pallas_call(
        _fused_body,
        grid_spec=grid_spec,
        out_shape=[
            jax.ShapeDtypeStruct((n, 8), jnp.float32),
            jax.ShapeDtypeStruct((n, 4), jnp.float32),
            jax.ShapeDtypeStruct((n, 4), jnp.int32),
            jax.ShapeDtypeStruct((1, 1), jnp.float32),
        ],
    )(opcode_idx.reshape(n, 1), a.reshape(n, 1), operand.reshape(n, 1),
      c_in.reshape(n, 1), opcode_embed, W_in, b_in.reshape(1, -1), Wr,
      br.reshape(1, -1), W1, b1, W2, b2, Wh1, bh1.reshape(1, -1), Wh2,
      bh2.reshape(1, -1), Wf1, bf1.reshape(1, -1), Wf2, bf2.reshape(1, -1))
    return rb, fl, ti, aux.reshape(())
